# baseline (speedup 1.0000x reference)
import functools

import jax
import jax.numpy as jnp
from jax import lax
from jax.experimental import pallas as pl
from jax.experimental.pallas import tpu as pltpu
from jax.experimental.pallas import tpu_sc as plsc

N_NODES = 10000
N_PAD = 10240
N_EDGES = 320000
NC = 2
NS = 16
NW = NC * NS
EPT = N_EDGES // NW
CHUNK = 128
NCHUNK = 80
IDXC = NCHUNK + 1
ROWS_PT = N_PAD // NS


def _mesh():
  return plsc.VectorSubcoreMesh(core_axis_name="c", subcore_axis_name="s",
                                num_cores=NC, num_subcores=NS)


@functools.cache
def _make_seg():
  scratch = [
      pltpu.VMEM_SHARED((N_PAD, 128), jnp.float32),
      pltpu.VMEM((2, CHUNK), jnp.int32),
      pltpu.VMEM((2, CHUNK), jnp.int32),
      pltpu.VMEM((CHUNK, 128), jnp.float32),
      pltpu.VMEM((CHUNK, 128), jnp.float32),
      pltpu.SemaphoreType.DMA,
      pltpu.SemaphoreType.DMA,
  ]

  def body(p_hbm, idx_hbm, zd_hbm, part_hbm,
           acc, ib0, ib1, rows0, rows1, sem0, sem1):
    c = lax.axis_index("c")
    s = lax.axis_index("s")
    wid = c * NS + s
    r0 = s * ROWS_PT
    pltpu.sync_copy(zd_hbm.at[pl.ds(0, CHUNK)], rows0)

    def zstep(j, carry):
      pltpu.sync_copy(rows0, acc.at[pl.ds(r0 + j * CHUNK, CHUNK)])
      return carry

    lax.fori_loop(0, ROWS_PT // CHUNK, zstep, 0)

    ibase = wid * IDXC
    pltpu.sync_copy(idx_hbm.at[ibase], ib0)
    pltpu.async_copy(p_hbm.at[ib0.at[0]], rows0, sem0)
    plsc.subcore_barrier()

    def step(g, carry):
      j = ibase + 2 * g
      pltpu.sync_copy(idx_hbm.at[j + 1], ib1)
      pltpu.async_copy(p_hbm.at[ib1.at[0]], rows1, sem1)
      pltpu.make_async_copy(p_hbm.at[ib0.at[0]], rows0, sem0).wait()
      pltpu.sync_copy(rows0, acc.at[ib0.at[1]], add=True)
      pltpu.sync_copy(idx_hbm.at[j + 2], ib0)
      pltpu.async_copy(p_hbm.at[ib0.at[0]], rows0, sem0)
      pltpu.make_async_copy(p_hbm.at[ib1.at[0]], rows1, sem1).wait()
      pltpu.sync_copy(rows1, acc.at[ib1.at[1]], add=True)
      return carry

    lax.fori_loop(0, NCHUNK // 2, step, 0)
    pltpu.make_async_copy(p_hbm.at[ib0.at[0]], rows0, sem0).wait()
    plsc.subcore_barrier()

    def wstep(j, carry):
      o = r0 + j * CHUNK
      pltpu.sync_copy(acc.at[pl.ds(o, CHUNK)], rows0)
      pltpu.sync_copy(rows0, part_hbm.at[c, pl.ds(o, CHUNK)])
      return carry

    lax.fori_loop(0, ROWS_PT // CHUNK, wstep, 0)

  return pl.kernel(
      body,
      out_type=jax.ShapeDtypeStruct((NC, N_PAD, 128), jnp.float32),
      mesh=_mesh(), scratch_types=scratch)


@functools.cache
def _make_cnt():
  scratch = [
      pltpu.VMEM_SHARED((N_PAD, 128), jnp.float32),
      pltpu.VMEM((2, CHUNK), jnp.int32),
      pltpu.VMEM((2, CHUNK), jnp.int32),
      pltpu.VMEM((CHUNK, 128), jnp.float32),
      pltpu.SemaphoreType.DMA,
  ]

  def body(idx_hbm, zd_hbm, ones_hbm, cnt_hbm, cacc, ib0, ib1, cbuf, semi):
    c = lax.axis_index("c")
    s = lax.axis_index("s")
    wid = c * NS + s
    r0 = s * ROWS_PT
    pltpu.sync_copy(zd_hbm.at[pl.ds(0, CHUNK)], cbuf)

    def zstep(j, carry):
      pltpu.sync_copy(cbuf, cacc.at[pl.ds(r0 + j * CHUNK, CHUNK)])
      return carry

    lax.fori_loop(0, ROWS_PT // CHUNK, zstep, 0)
    pltpu.sync_copy(ones_hbm, cbuf)
    ibase = wid * IDXC
    pltpu.sync_copy(idx_hbm.at[ibase], ib0)
    plsc.subcore_barrier()

    def step(g, carry):
      j = ibase + 2 * g
      pltpu.async_copy(idx_hbm.at[j + 1], ib1, semi)
      pltpu.sync_copy(cbuf, cacc.at[ib0.at[1]], add=True)
      pltpu.make_async_copy(idx_hbm.at[j + 1], ib1, semi).wait()
      pltpu.async_copy(idx_hbm.at[j + 2], ib0, semi)
      pltpu.sync_copy(cbuf, cacc.at[ib1.at[1]], add=True)
      pltpu.make_async_copy(idx_hbm.at[j + 2], ib0, semi).wait()
      return carry

    lax.fori_loop(0, NCHUNK // 2, step, 0)
    plsc.subcore_barrier()

    def wstep(j, carry):
      o = r0 + j * CHUNK
      pltpu.sync_copy(cacc.at[pl.ds(o, CHUNK)], cbuf)
      pltpu.sync_copy(cbuf, cnt_hbm.at[c, pl.ds(o, CHUNK)])
      return carry

    lax.fori_loop(0, ROWS_PT // CHUNK, wstep, 0)

  return pl.kernel(
      body,
      out_type=jax.ShapeDtypeStruct((NC, N_PAD, 128), jnp.float32),
      mesh=_mesh(), scratch_types=scratch)


def _seg128(*args):
  return _make_seg()(*args)


def _cnt128(*args):
  return _make_cnt()(*args)


_GRID = 10
_R = N_PAD // _GRID


def _proj_body(x_ref, wl_ref, wr_ref, b_ref, p_ref, q_ref):
  xb = x_ref[...]
  p_ref[...] = jnp.dot(xb, wl_ref[...], preferred_element_type=jnp.float32)
  q_ref[...] = (jnp.dot(xb, wr_ref[...], preferred_element_type=jnp.float32)
                + b_ref[...])


def _combine_body(part_ref, cnt_ref, q_ref, wl_ref, wr_ref, b_ref,
                  p_ref, qn_ref):
  agg = part_ref[0] + part_ref[1]
  cnt = cnt_ref[0, :, 0:1] + cnt_ref[1, :, 0:1]
  inv = 1.0 / jnp.maximum(cnt, 1.0)
  h = jnp.maximum(agg * inv + q_ref[...], 0.0)
  p_ref[...] = jnp.dot(h, wl_ref[...], preferred_element_type=jnp.float32)
  qn_ref[...] = (jnp.dot(h, wr_ref[...], preferred_element_type=jnp.float32)
                 + b_ref[...])


def _last_body(part_ref, cnt_ref, q_ref, wr_ref, b_ref, h_ref, qn_ref):
  agg = part_ref[0] + part_ref[1]
  cnt = cnt_ref[0, :, 0:1] + cnt_ref[1, :, 0:1]
  inv = 1.0 / jnp.maximum(cnt, 1.0)
  h = jnp.maximum(agg * inv + q_ref[...], 0.0)
  h_ref[...] = h
  qn_ref[...] = (jnp.dot(h, wr_ref[...], preferred_element_type=jnp.float32)
                 + b_ref[...])


def _final_body(part_ref, cnt_ref, q_ref, wl_ref, o_ref):
  agg = part_ref[0] + part_ref[1]
  cnt = cnt_ref[0, :, 0:1] + cnt_ref[1, :, 0:1]
  inv = 1.0 / jnp.maximum(cnt, 1.0)
  h = (jnp.dot(agg * inv, wl_ref[...], preferred_element_type=jnp.float32)
       + q_ref[...])
  n = jnp.sqrt(jnp.sum(h * h, axis=1, keepdims=True))
  o_ref[...] = h / jnp.maximum(n, 1e-12)


def _proj_call(xp, wlT, wrT, b, dout):
  return pl.pallas_call(
      _proj_body,
      grid=(_GRID,),
      in_specs=[
          pl.BlockSpec((_R, 128), lambda i: (i, 0)),
          pl.BlockSpec((128, dout), lambda i: (0, 0)),
          pl.BlockSpec((128, dout), lambda i: (0, 0)),
          pl.BlockSpec((1, dout), lambda i: (0, 0)),
      ],
      out_specs=[
          pl.BlockSpec((_R, dout), lambda i: (i, 0)),
          pl.BlockSpec((_R, dout), lambda i: (i, 0)),
      ],
      out_shape=[
          jax.ShapeDtypeStruct((N_PAD, dout), jnp.float32),
          jax.ShapeDtypeStruct((N_PAD, dout), jnp.float32),
      ],
  )(xp, wlT, wrT, b)


def _combine_call(part, cnt, q, wlT, wrT, b, din, dout):
  return pl.pallas_call(
      _combine_body,
      grid=(_GRID,),
      in_specs=[
          pl.BlockSpec((NC, _R, din), lambda i: (0, i, 0)),
          pl.BlockSpec((NC, _R, 128), lambda i: (0, i, 0)),
          pl.BlockSpec((_R, din), lambda i: (i, 0)),
          pl.BlockSpec((din, dout), lambda i: (0, 0)),
          pl.BlockSpec((din, dout), lambda i: (0, 0)),
          pl.BlockSpec((1, dout), lambda i: (0, 0)),
      ],
      out_specs=[
          pl.BlockSpec((_R, dout), lambda i: (i, 0)),
          pl.BlockSpec((_R, dout), lambda i: (i, 0)),
      ],
      out_shape=[
          jax.ShapeDtypeStruct((N_PAD, dout), jnp.float32),
          jax.ShapeDtypeStruct((N_PAD, dout), jnp.float32),
      ],
  )(part, cnt, q, wlT, wrT, b)


def _last_call(part, cnt, q, wrT, b):
  return pl.pallas_call(
      _last_body,
      grid=(_GRID,),
      in_specs=[
          pl.BlockSpec((NC, _R, 128), lambda i: (0, i, 0)),
          pl.BlockSpec((NC, _R, 128), lambda i: (0, i, 0)),
          pl.BlockSpec((_R, 128), lambda i: (i, 0)),
          pl.BlockSpec((128, 64), lambda i: (0, 0)),
          pl.BlockSpec((1, 64), lambda i: (0, 0)),
      ],
      out_specs=[
          pl.BlockSpec((_R, 128), lambda i: (i, 0)),
          pl.BlockSpec((_R, 64), lambda i: (i, 0)),
      ],
      out_shape=[
          jax.ShapeDtypeStruct((N_PAD, 128), jnp.float32),
          jax.ShapeDtypeStruct((N_PAD, 64), jnp.float32),
      ],
  )(part, cnt, q, wrT, b)


def _final_call(part, cnt, q, wlT):
  return pl.pallas_call(
      _final_body,
      grid=(_GRID,),
      in_specs=[
          pl.BlockSpec((NC, _R, 128), lambda i: (0, i, 0)),
          pl.BlockSpec((NC, _R, 128), lambda i: (0, i, 0)),
          pl.BlockSpec((_R, 64), lambda i: (i, 0)),
          pl.BlockSpec((128, 64), lambda i: (0, 0)),
      ],
      out_specs=pl.BlockSpec((_R, 64), lambda i: (i, 0)),
      out_shape=jax.ShapeDtypeStruct((N_PAD, 64), jnp.float32),
  )(part, cnt, q, wlT)


@jax.jit
def kernel(x, edge_index, W1l, b1l, W1r, W2l, b2l, W2r, W3l, b3l, W3r):
  xp = jnp.pad(x, ((0, N_PAD - N_NODES), (0, 0)))
  pad_w = IDXC * CHUNK - EPT
  ei = jnp.reshape(edge_index, (2, NW, EPT))
  ei = jnp.pad(ei, ((0, 0), (0, 0), (0, pad_w)),
               constant_values=N_PAD - 1)
  packed = jnp.reshape(
      jnp.transpose(jnp.reshape(ei, (2, NW, IDXC, CHUNK)), (1, 2, 0, 3)),
      (NW * IDXC, 2, CHUNK))
  z128 = jnp.zeros((N_PAD, 128), jnp.float32)
  ones = jnp.ones((CHUNK, 128), jnp.float32)

  cnt = _cnt128(packed, z128, ones)
  p1, q1 = _proj_call(xp, W1l.T, W1r.T, b1l[None], 128)
  part1 = _seg128(p1, packed, z128)
  p2, q2 = _combine_call(part1, cnt, q1, W2l.T, W2r.T, b2l[None], 128, 128)
  part2 = _seg128(p2, packed, z128)
  h2, q3 = _last_call(part2, cnt, q2, W3r.T, b3l[None])
  part3 = _seg128(h2, packed, z128)
  out = _final_call(part3, cnt, q3, W3l.T)
  return out[:N_NODES]


# speedup vs baseline: 1.1301x; 1.1301x over previous
import functools

import jax
import jax.numpy as jnp
from jax import lax
from jax.experimental import pallas as pl
from jax.experimental.pallas import tpu as pltpu
from jax.experimental.pallas import tpu_sc as plsc

N_NODES = 10000
N_PAD = 10240
N_EDGES = 320000
NC = 2
NS = 16
NW = NC * NS
EPT = N_EDGES // NW
CHUNK = 128
NCHUNK = 80
IDXC = NCHUNK + 1
ROWS_PT = N_PAD // NS


def _mesh():
  return plsc.VectorSubcoreMesh(core_axis_name="c", subcore_axis_name="s",
                                num_cores=NC, num_subcores=NS)


@functools.cache
def _make_seg():
  scratch = [
      pltpu.VMEM_SHARED((N_PAD, 128), jnp.float32),
      pltpu.VMEM((2, CHUNK), jnp.int32),
      pltpu.VMEM((2, CHUNK), jnp.int32),
      pltpu.VMEM((CHUNK, 128), jnp.float32),
      pltpu.VMEM((CHUNK, 128), jnp.float32),
      pltpu.SemaphoreType.DMA,
      pltpu.SemaphoreType.DMA,
  ]

  def body(p_hbm, idx_hbm, zd_hbm, part_hbm,
           acc, ib0, ib1, rows0, rows1, sem0, sem1):
    c = lax.axis_index("c")
    s = lax.axis_index("s")
    wid = c * NS + s
    r0 = s * ROWS_PT
    pltpu.sync_copy(zd_hbm.at[pl.ds(0, CHUNK)], rows0)

    def zstep(j, carry):
      pltpu.sync_copy(rows0, acc.at[pl.ds(r0 + j * CHUNK, CHUNK)])
      return carry

    lax.fori_loop(0, ROWS_PT // CHUNK, zstep, 0)

    ibase = wid * IDXC
    plsc.subcore_barrier()

    def step(i, carry):
      pltpu.sync_copy(idx_hbm.at[ibase + i], ib0)
      pltpu.async_copy(p_hbm.at[ib0.at[0]], rows0, sem0).wait()
      pltpu.sync_copy(rows0, acc.at[ib0.at[1]], add=True)
      return carry

    lax.fori_loop(0, NCHUNK, step, 0)
    plsc.subcore_barrier()

    def wstep(j, carry):
      o = r0 + j * CHUNK
      pltpu.sync_copy(acc.at[pl.ds(o, CHUNK)], rows0)
      pltpu.sync_copy(rows0, part_hbm.at[c, pl.ds(o, CHUNK)])
      return carry

    lax.fori_loop(0, ROWS_PT // CHUNK, wstep, 0)

  return pl.kernel(
      body,
      out_type=jax.ShapeDtypeStruct((NC, N_PAD, 128), jnp.float32),
      mesh=_mesh(), scratch_types=scratch)


@functools.cache
def _make_cnt():
  scratch = [
      pltpu.VMEM_SHARED((N_PAD, 128), jnp.float32),
      pltpu.VMEM((2, CHUNK), jnp.int32),
      pltpu.VMEM((2, CHUNK), jnp.int32),
      pltpu.VMEM((CHUNK, 128), jnp.float32),
      pltpu.SemaphoreType.DMA,
  ]

  def body(idx_hbm, zd_hbm, ones_hbm, cnt_hbm, cacc, ib0, ib1, cbuf, semi):
    c = lax.axis_index("c")
    s = lax.axis_index("s")
    wid = c * NS + s
    r0 = s * ROWS_PT
    pltpu.sync_copy(zd_hbm.at[pl.ds(0, CHUNK)], cbuf)

    def zstep(j, carry):
      pltpu.sync_copy(cbuf, cacc.at[pl.ds(r0 + j * CHUNK, CHUNK)])
      return carry

    lax.fori_loop(0, ROWS_PT // CHUNK, zstep, 0)
    pltpu.sync_copy(ones_hbm, cbuf)
    ibase = wid * IDXC
    pltpu.sync_copy(idx_hbm.at[ibase], ib0)
    plsc.subcore_barrier()

    def step(g, carry):
      j = ibase + 2 * g
      pltpu.async_copy(idx_hbm.at[j + 1], ib1, semi)
      pltpu.sync_copy(cbuf, cacc.at[ib0.at[1]], add=True)
      pltpu.make_async_copy(idx_hbm.at[j + 1], ib1, semi).wait()
      pltpu.async_copy(idx_hbm.at[j + 2], ib0, semi)
      pltpu.sync_copy(cbuf, cacc.at[ib1.at[1]], add=True)
      pltpu.make_async_copy(idx_hbm.at[j + 2], ib0, semi).wait()
      return carry

    lax.fori_loop(0, NCHUNK // 2, step, 0)
    plsc.subcore_barrier()

    def wstep(j, carry):
      o = r0 + j * CHUNK
      pltpu.sync_copy(cacc.at[pl.ds(o, CHUNK)], cbuf)
      pltpu.sync_copy(cbuf, cnt_hbm.at[c, pl.ds(o, CHUNK)])
      return carry

    lax.fori_loop(0, ROWS_PT // CHUNK, wstep, 0)

  return pl.kernel(
      body,
      out_type=jax.ShapeDtypeStruct((NC, N_PAD, 128), jnp.float32),
      mesh=_mesh(), scratch_types=scratch)


def _seg128(*args):
  return _make_seg()(*args)


def _cnt128(*args):
  return _make_cnt()(*args)


_GRID = 10
_R = N_PAD // _GRID


def _proj_body(x_ref, wl_ref, wr_ref, b_ref, p_ref, q_ref):
  xb = x_ref[...]
  p_ref[...] = jnp.dot(xb, wl_ref[...], preferred_element_type=jnp.float32)
  q_ref[...] = (jnp.dot(xb, wr_ref[...], preferred_element_type=jnp.float32)
                + b_ref[...])


def _combine_body(part_ref, cnt_ref, q_ref, wl_ref, wr_ref, b_ref,
                  p_ref, qn_ref):
  agg = part_ref[0] + part_ref[1]
  cnt = cnt_ref[0, :, 0:1] + cnt_ref[1, :, 0:1]
  inv = 1.0 / jnp.maximum(cnt, 1.0)
  h = jnp.maximum(agg * inv + q_ref[...], 0.0)
  p_ref[...] = jnp.dot(h, wl_ref[...], preferred_element_type=jnp.float32)
  qn_ref[...] = (jnp.dot(h, wr_ref[...], preferred_element_type=jnp.float32)
                 + b_ref[...])


def _last_body(part_ref, cnt_ref, q_ref, wr_ref, b_ref, h_ref, qn_ref):
  agg = part_ref[0] + part_ref[1]
  cnt = cnt_ref[0, :, 0:1] + cnt_ref[1, :, 0:1]
  inv = 1.0 / jnp.maximum(cnt, 1.0)
  h = jnp.maximum(agg * inv + q_ref[...], 0.0)
  h_ref[...] = h
  qn_ref[...] = (jnp.dot(h, wr_ref[...], preferred_element_type=jnp.float32)
                 + b_ref[...])


def _final_body(part_ref, cnt_ref, q_ref, wl_ref, o_ref):
  agg = part_ref[0] + part_ref[1]
  cnt = cnt_ref[0, :, 0:1] + cnt_ref[1, :, 0:1]
  inv = 1.0 / jnp.maximum(cnt, 1.0)
  h = (jnp.dot(agg * inv, wl_ref[...], preferred_element_type=jnp.float32)
       + q_ref[...])
  n = jnp.sqrt(jnp.sum(h * h, axis=1, keepdims=True))
  o_ref[...] = h / jnp.maximum(n, 1e-12)


def _proj_call(xp, wlT, wrT, b, dout):
  return pl.pallas_call(
      _proj_body,
      grid=(_GRID,),
      in_specs=[
          pl.BlockSpec((_R, 128), lambda i: (i, 0)),
          pl.BlockSpec((128, dout), lambda i: (0, 0)),
          pl.BlockSpec((128, dout), lambda i: (0, 0)),
          pl.BlockSpec((1, dout), lambda i: (0, 0)),
      ],
      out_specs=[
          pl.BlockSpec((_R, dout), lambda i: (i, 0)),
          pl.BlockSpec((_R, dout), lambda i: (i, 0)),
      ],
      out_shape=[
          jax.ShapeDtypeStruct((N_PAD, dout), jnp.float32),
          jax.ShapeDtypeStruct((N_PAD, dout), jnp.float32),
      ],
  )(xp, wlT, wrT, b)


def _combine_call(part, cnt, q, wlT, wrT, b, din, dout):
  return pl.pallas_call(
      _combine_body,
      grid=(_GRID,),
      in_specs=[
          pl.BlockSpec((NC, _R, din), lambda i: (0, i, 0)),
          pl.BlockSpec((NC, _R, 128), lambda i: (0, i, 0)),
          pl.BlockSpec((_R, din), lambda i: (i, 0)),
          pl.BlockSpec((din, dout), lambda i: (0, 0)),
          pl.BlockSpec((din, dout), lambda i: (0, 0)),
          pl.BlockSpec((1, dout), lambda i: (0, 0)),
      ],
      out_specs=[
          pl.BlockSpec((_R, dout), lambda i: (i, 0)),
          pl.BlockSpec((_R, dout), lambda i: (i, 0)),
      ],
      out_shape=[
          jax.ShapeDtypeStruct((N_PAD, dout), jnp.float32),
          jax.ShapeDtypeStruct((N_PAD, dout), jnp.float32),
      ],
  )(part, cnt, q, wlT, wrT, b)


def _last_call(part, cnt, q, wrT, b):
  return pl.pallas_call(
      _last_body,
      grid=(_GRID,),
      in_specs=[
          pl.BlockSpec((NC, _R, 128), lambda i: (0, i, 0)),
          pl.BlockSpec((NC, _R, 128), lambda i: (0, i, 0)),
          pl.BlockSpec((_R, 128), lambda i: (i, 0)),
          pl.BlockSpec((128, 64), lambda i: (0, 0)),
          pl.BlockSpec((1, 64), lambda i: (0, 0)),
      ],
      out_specs=[
          pl.BlockSpec((_R, 128), lambda i: (i, 0)),
          pl.BlockSpec((_R, 64), lambda i: (i, 0)),
      ],
      out_shape=[
          jax.ShapeDtypeStruct((N_PAD, 128), jnp.float32),
          jax.ShapeDtypeStruct((N_PAD, 64), jnp.float32),
      ],
  )(part, cnt, q, wrT, b)


def _final_call(part, cnt, q, wlT):
  return pl.pallas_call(
      _final_body,
      grid=(_GRID,),
      in_specs=[
          pl.BlockSpec((NC, _R, 128), lambda i: (0, i, 0)),
          pl.BlockSpec((NC, _R, 128), lambda i: (0, i, 0)),
          pl.BlockSpec((_R, 64), lambda i: (i, 0)),
          pl.BlockSpec((128, 64), lambda i: (0, 0)),
      ],
      out_specs=pl.BlockSpec((_R, 64), lambda i: (i, 0)),
      out_shape=jax.ShapeDtypeStruct((N_PAD, 64), jnp.float32),
  )(part, cnt, q, wlT)


@jax.jit
def kernel(x, edge_index, W1l, b1l, W1r, W2l, b2l, W2r, W3l, b3l, W3r):
  xp = jnp.pad(x, ((0, N_PAD - N_NODES), (0, 0)))
  pad_w = IDXC * CHUNK - EPT
  ei = jnp.reshape(edge_index, (2, NW, EPT))
  ei = jnp.pad(ei, ((0, 0), (0, 0), (0, pad_w)),
               constant_values=N_PAD - 1)
  packed = jnp.reshape(
      jnp.transpose(jnp.reshape(ei, (2, NW, IDXC, CHUNK)), (1, 2, 0, 3)),
      (NW * IDXC, 2, CHUNK))
  z128 = jnp.zeros((N_PAD, 128), jnp.float32)
  ones = jnp.ones((CHUNK, 128), jnp.float32)

  cnt = _cnt128(packed, z128, ones)
  p1, q1 = _proj_call(xp, W1l.T, W1r.T, b1l[None], 128)
  part1 = _seg128(p1, packed, z128)
  p2, q2 = _combine_call(part1, cnt, q1, W2l.T, W2r.T, b2l[None], 128, 128)
  part2 = _seg128(p2, packed, z128)
  h2, q3 = _last_call(part2, cnt, q2, W3r.T, b3l[None])
  part3 = _seg128(h2, packed, z128)
  out = _final_call(part3, cnt, q3, W3l.T)
  return out[:N_NODES]


# speedup vs baseline: 1.5223x; 1.3471x over previous
import functools

import jax
import jax.numpy as jnp
from jax import lax
from jax.experimental import pallas as pl
from jax.experimental.pallas import tpu as pltpu
from jax.experimental.pallas import tpu_sc as plsc

N_NODES = 10000
N_PAD = 10240
N_EDGES = 320000
NC = 2
NS = 16
NW = NC * NS
EPT = N_EDGES // NW
CHUNK = 80
NCHUNK = 126
IDXC = NCHUNK + 1
ROWS_PT = N_PAD // NS


def _mesh():
  return plsc.VectorSubcoreMesh(core_axis_name="c", subcore_axis_name="s",
                                num_cores=NC, num_subcores=NS)


@functools.cache
def _make_seg():
  scratch = [
      pltpu.VMEM_SHARED((N_PAD, 128), jnp.float32),
      pltpu.VMEM((2, CHUNK), jnp.int32),
      pltpu.VMEM((2, CHUNK), jnp.int32),
      pltpu.VMEM((CHUNK, 128), jnp.float32),
      pltpu.VMEM((CHUNK, 128), jnp.float32),
      pltpu.SemaphoreType.DMA,
      pltpu.SemaphoreType.DMA,
  ]

  def body(p_hbm, idx_hbm, zd_hbm, part_hbm,
           acc, ib0, ib1, rows0, rows1, sem0, sem1):
    c = lax.axis_index("c")
    s = lax.axis_index("s")
    wid = c * NS + s
    r0 = s * ROWS_PT
    pltpu.sync_copy(zd_hbm.at[pl.ds(0, CHUNK)], rows0)

    def zstep(j, carry):
      pltpu.sync_copy(rows0, acc.at[pl.ds(r0 + j * CHUNK, CHUNK)])
      return carry

    lax.fori_loop(0, ROWS_PT // CHUNK, zstep, 0)

    ibase = wid * IDXC
    plsc.subcore_barrier()

    def step(i, carry):
      pltpu.sync_copy(idx_hbm.at[ibase + i], ib0)
      pltpu.async_copy(p_hbm.at[ib0.at[0]], rows0, sem0).wait()
      pltpu.sync_copy(rows0, acc.at[ib0.at[1]], add=True)
      return carry

    lax.fori_loop(0, NCHUNK, step, 0)
    plsc.subcore_barrier()

    def wstep(j, carry):
      o = r0 + j * CHUNK
      pltpu.sync_copy(acc.at[pl.ds(o, CHUNK)], rows0)
      pltpu.sync_copy(rows0, part_hbm.at[c, pl.ds(o, CHUNK)])
      return carry

    lax.fori_loop(0, ROWS_PT // CHUNK, wstep, 0)

  return pl.kernel(
      body,
      out_type=jax.ShapeDtypeStruct((NC, N_PAD, 128), jnp.float32),
      mesh=_mesh(), scratch_types=scratch)


@functools.cache
def _make_cnt():
  scratch = [
      pltpu.VMEM_SHARED((N_PAD, 128), jnp.float32),
      pltpu.VMEM((2, CHUNK), jnp.int32),
      pltpu.VMEM((2, CHUNK), jnp.int32),
      pltpu.VMEM((CHUNK, 128), jnp.float32),
      pltpu.SemaphoreType.DMA,
  ]

  def body(idx_hbm, zd_hbm, ones_hbm, cnt_hbm, cacc, ib0, ib1, cbuf, semi):
    c = lax.axis_index("c")
    s = lax.axis_index("s")
    wid = c * NS + s
    r0 = s * ROWS_PT
    pltpu.sync_copy(zd_hbm.at[pl.ds(0, CHUNK)], cbuf)

    def zstep(j, carry):
      pltpu.sync_copy(cbuf, cacc.at[pl.ds(r0 + j * CHUNK, CHUNK)])
      return carry

    lax.fori_loop(0, ROWS_PT // CHUNK, zstep, 0)
    pltpu.sync_copy(ones_hbm, cbuf)
    ibase = wid * IDXC
    pltpu.sync_copy(idx_hbm.at[ibase], ib0)
    plsc.subcore_barrier()

    def step(g, carry):
      j = ibase + 2 * g
      pltpu.async_copy(idx_hbm.at[j + 1], ib1, semi)
      pltpu.sync_copy(cbuf, cacc.at[ib0.at[1]], add=True)
      pltpu.make_async_copy(idx_hbm.at[j + 1], ib1, semi).wait()
      pltpu.async_copy(idx_hbm.at[j + 2], ib0, semi)
      pltpu.sync_copy(cbuf, cacc.at[ib1.at[1]], add=True)
      pltpu.make_async_copy(idx_hbm.at[j + 2], ib0, semi).wait()
      return carry

    lax.fori_loop(0, NCHUNK // 2, step, 0)
    plsc.subcore_barrier()

    def wstep(j, carry):
      o = r0 + j * CHUNK
      pltpu.sync_copy(cacc.at[pl.ds(o, CHUNK)], cbuf)
      pltpu.sync_copy(cbuf, cnt_hbm.at[c, pl.ds(o, CHUNK)])
      return carry

    lax.fori_loop(0, ROWS_PT // CHUNK, wstep, 0)

  return pl.kernel(
      body,
      out_type=jax.ShapeDtypeStruct((NC, N_PAD, 128), jnp.float32),
      mesh=_mesh(), scratch_types=scratch)


def _seg128(*args):
  return _make_seg()(*args)


def _cnt128(*args):
  return _make_cnt()(*args)


_GRID = 10
_R = N_PAD // _GRID


def _proj_body(x_ref, wl_ref, wr_ref, b_ref, p_ref, q_ref):
  xb = x_ref[...]
  p_ref[...] = jnp.dot(xb, wl_ref[...], preferred_element_type=jnp.float32)
  q_ref[...] = (jnp.dot(xb, wr_ref[...], preferred_element_type=jnp.float32)
                + b_ref[...])


def _combine_body(part_ref, cnt_ref, q_ref, wl_ref, wr_ref, b_ref,
                  p_ref, qn_ref):
  agg = part_ref[0] + part_ref[1]
  cnt = cnt_ref[0, :, 0:1] + cnt_ref[1, :, 0:1]
  inv = 1.0 / jnp.maximum(cnt, 1.0)
  h = jnp.maximum(agg * inv + q_ref[...], 0.0)
  p_ref[...] = jnp.dot(h, wl_ref[...], preferred_element_type=jnp.float32)
  qn_ref[...] = (jnp.dot(h, wr_ref[...], preferred_element_type=jnp.float32)
                 + b_ref[...])


def _last_body(part_ref, cnt_ref, q_ref, wr_ref, b_ref, h_ref, qn_ref):
  agg = part_ref[0] + part_ref[1]
  cnt = cnt_ref[0, :, 0:1] + cnt_ref[1, :, 0:1]
  inv = 1.0 / jnp.maximum(cnt, 1.0)
  h = jnp.maximum(agg * inv + q_ref[...], 0.0)
  h_ref[...] = h
  qn_ref[...] = (jnp.dot(h, wr_ref[...], preferred_element_type=jnp.float32)
                 + b_ref[...])


def _final_body(part_ref, cnt_ref, q_ref, wl_ref, o_ref):
  agg = part_ref[0] + part_ref[1]
  cnt = cnt_ref[0, :, 0:1] + cnt_ref[1, :, 0:1]
  inv = 1.0 / jnp.maximum(cnt, 1.0)
  h = (jnp.dot(agg * inv, wl_ref[...], preferred_element_type=jnp.float32)
       + q_ref[...])
  n = jnp.sqrt(jnp.sum(h * h, axis=1, keepdims=True))
  o_ref[...] = h / jnp.maximum(n, 1e-12)


def _proj_call(xp, wlT, wrT, b, dout):
  return pl.pallas_call(
      _proj_body,
      grid=(_GRID,),
      in_specs=[
          pl.BlockSpec((_R, 128), lambda i: (i, 0)),
          pl.BlockSpec((128, dout), lambda i: (0, 0)),
          pl.BlockSpec((128, dout), lambda i: (0, 0)),
          pl.BlockSpec((1, dout), lambda i: (0, 0)),
      ],
      out_specs=[
          pl.BlockSpec((_R, dout), lambda i: (i, 0)),
          pl.BlockSpec((_R, dout), lambda i: (i, 0)),
      ],
      out_shape=[
          jax.ShapeDtypeStruct((N_PAD, dout), jnp.float32),
          jax.ShapeDtypeStruct((N_PAD, dout), jnp.float32),
      ],
  )(xp, wlT, wrT, b)


def _combine_call(part, cnt, q, wlT, wrT, b, din, dout):
  return pl.pallas_call(
      _combine_body,
      grid=(_GRID,),
      in_specs=[
          pl.BlockSpec((NC, _R, din), lambda i: (0, i, 0)),
          pl.BlockSpec((NC, _R, 128), lambda i: (0, i, 0)),
          pl.BlockSpec((_R, din), lambda i: (i, 0)),
          pl.BlockSpec((din, dout), lambda i: (0, 0)),
          pl.BlockSpec((din, dout), lambda i: (0, 0)),
          pl.BlockSpec((1, dout), lambda i: (0, 0)),
      ],
      out_specs=[
          pl.BlockSpec((_R, dout), lambda i: (i, 0)),
          pl.BlockSpec((_R, dout), lambda i: (i, 0)),
      ],
      out_shape=[
          jax.ShapeDtypeStruct((N_PAD, dout), jnp.float32),
          jax.ShapeDtypeStruct((N_PAD, dout), jnp.float32),
      ],
  )(part, cnt, q, wlT, wrT, b)


def _last_call(part, cnt, q, wrT, b):
  return pl.pallas_call(
      _last_body,
      grid=(_GRID,),
      in_specs=[
          pl.BlockSpec((NC, _R, 128), lambda i: (0, i, 0)),
          pl.BlockSpec((NC, _R, 128), lambda i: (0, i, 0)),
          pl.BlockSpec((_R, 128), lambda i: (i, 0)),
          pl.BlockSpec((128, 64), lambda i: (0, 0)),
          pl.BlockSpec((1, 64), lambda i: (0, 0)),
      ],
      out_specs=[
          pl.BlockSpec((_R, 128), lambda i: (i, 0)),
          pl.BlockSpec((_R, 64), lambda i: (i, 0)),
      ],
      out_shape=[
          jax.ShapeDtypeStruct((N_PAD, 128), jnp.float32),
          jax.ShapeDtypeStruct((N_PAD, 64), jnp.float32),
      ],
  )(part, cnt, q, wrT, b)


def _final_call(part, cnt, q, wlT):
  return pl.pallas_call(
      _final_body,
      grid=(_GRID,),
      in_specs=[
          pl.BlockSpec((NC, _R, 128), lambda i: (0, i, 0)),
          pl.BlockSpec((NC, _R, 128), lambda i: (0, i, 0)),
          pl.BlockSpec((_R, 64), lambda i: (i, 0)),
          pl.BlockSpec((128, 64), lambda i: (0, 0)),
      ],
      out_specs=pl.BlockSpec((_R, 64), lambda i: (i, 0)),
      out_shape=jax.ShapeDtypeStruct((N_PAD, 64), jnp.float32),
  )(part, cnt, q, wlT)


@jax.jit
def kernel(x, edge_index, W1l, b1l, W1r, W2l, b2l, W2r, W3l, b3l, W3r):
  xp = jnp.pad(x, ((0, N_PAD - N_NODES), (0, 0)))
  pad_w = IDXC * CHUNK - EPT
  ei = jnp.reshape(edge_index, (2, NW, EPT))
  ei = jnp.pad(ei, ((0, 0), (0, 0), (0, pad_w)),
               constant_values=N_PAD - 1)
  packed = jnp.reshape(
      jnp.transpose(jnp.reshape(ei, (2, NW, IDXC, CHUNK)), (1, 2, 0, 3)),
      (NW * IDXC, 2, CHUNK))
  z128 = jnp.zeros((N_PAD, 128), jnp.float32)
  ones = jnp.ones((CHUNK, 128), jnp.float32)

  cnt = _cnt128(packed, z128, ones)
  p1, q1 = _proj_call(xp, W1l.T, W1r.T, b1l[None], 128)
  part1 = _seg128(p1, packed, z128)
  p2, q2 = _combine_call(part1, cnt, q1, W2l.T, W2r.T, b2l[None], 128, 128)
  part2 = _seg128(p2, packed, z128)
  h2, q3 = _last_call(part2, cnt, q2, W3r.T, b3l[None])
  part3 = _seg128(h2, packed, z128)
  out = _final_call(part3, cnt, q3, W3l.T)
  return out[:N_NODES]


# speedup vs baseline: 1.7697x; 1.1625x over previous
import functools

import jax
import jax.numpy as jnp
from jax import lax
from jax.experimental import pallas as pl
from jax.experimental.pallas import tpu as pltpu
from jax.experimental.pallas import tpu_sc as plsc

N_NODES = 10000
N_PAD = 10240
N_EDGES = 320000
NC = 2
NS = 16
NW = NC * NS
EPT = N_EDGES // NW
CHUNK = 80
NCHUNK = 126
IDXC = NCHUNK + 1
ROWS_PT = N_PAD // NS


def _mesh():
  return plsc.VectorSubcoreMesh(core_axis_name="c", subcore_axis_name="s",
                                num_cores=NC, num_subcores=NS)


@functools.cache
def _make_seg():
  scratch = [
      pltpu.VMEM_SHARED((N_PAD, 128), jnp.float32),
      pltpu.VMEM((IDXC, 2, CHUNK), jnp.int32),
      pltpu.VMEM((CHUNK, 128), jnp.float32),
      pltpu.VMEM((CHUNK, 128), jnp.float32),
      pltpu.SemaphoreType.DMA,
      pltpu.SemaphoreType.DMA,
  ]

  def body(p_hbm, idx_hbm, zd_hbm, part_hbm,
           acc, ib, rows0, rows1, sem0, sem1):
    c = lax.axis_index("c")
    s = lax.axis_index("s")
    wid = c * NS + s
    r0 = s * ROWS_PT
    pltpu.sync_copy(idx_hbm.at[pl.ds(wid * IDXC, IDXC)], ib)
    pltpu.sync_copy(zd_hbm.at[pl.ds(0, CHUNK)], rows0)

    def zstep(j, carry):
      pltpu.sync_copy(rows0, acc.at[pl.ds(r0 + j * CHUNK, CHUNK)])
      return carry

    lax.fori_loop(0, ROWS_PT // CHUNK, zstep, 0)
    plsc.subcore_barrier()

    def step(i, carry):
      pltpu.async_copy(p_hbm.at[ib.at[i, 0]], rows0, sem0).wait()
      pltpu.sync_copy(rows0, acc.at[ib.at[i, 1]], add=True)
      return carry

    lax.fori_loop(0, NCHUNK, step, 0)
    plsc.subcore_barrier()

    def wstep(j, carry):
      o = r0 + j * CHUNK
      pltpu.sync_copy(acc.at[pl.ds(o, CHUNK)], rows0)
      pltpu.sync_copy(rows0, part_hbm.at[c, pl.ds(o, CHUNK)])
      return carry

    lax.fori_loop(0, ROWS_PT // CHUNK, wstep, 0)

  return pl.kernel(
      body,
      out_type=jax.ShapeDtypeStruct((NC, N_PAD, 128), jnp.float32),
      mesh=_mesh(), scratch_types=scratch)


@functools.cache
def _make_cnt():
  scratch = [
      pltpu.VMEM_SHARED((N_PAD, 128), jnp.float32),
      pltpu.VMEM((IDXC, 2, CHUNK), jnp.int32),
      pltpu.VMEM((CHUNK, 128), jnp.float32),
  ]

  def body(idx_hbm, zd_hbm, ones_hbm, cnt_hbm, cacc, ib, cbuf):
    c = lax.axis_index("c")
    s = lax.axis_index("s")
    wid = c * NS + s
    r0 = s * ROWS_PT
    pltpu.sync_copy(idx_hbm.at[pl.ds(wid * IDXC, IDXC)], ib)
    pltpu.sync_copy(zd_hbm.at[pl.ds(0, CHUNK)], cbuf)

    def zstep(j, carry):
      pltpu.sync_copy(cbuf, cacc.at[pl.ds(r0 + j * CHUNK, CHUNK)])
      return carry

    lax.fori_loop(0, ROWS_PT // CHUNK, zstep, 0)
    pltpu.sync_copy(ones_hbm, cbuf)
    plsc.subcore_barrier()

    def step(i, carry):
      pltpu.sync_copy(cbuf, cacc.at[ib.at[i, 1]], add=True)
      return carry

    lax.fori_loop(0, NCHUNK, step, 0)
    plsc.subcore_barrier()

    def wstep(j, carry):
      o = r0 + j * CHUNK
      pltpu.sync_copy(cacc.at[pl.ds(o, CHUNK)], cbuf)
      pltpu.sync_copy(cbuf, cnt_hbm.at[c, pl.ds(o, CHUNK)])
      return carry

    lax.fori_loop(0, ROWS_PT // CHUNK, wstep, 0)

  return pl.kernel(
      body,
      out_type=jax.ShapeDtypeStruct((NC, N_PAD, 128), jnp.float32),
      mesh=_mesh(), scratch_types=scratch)


def _seg128(*args):
  return _make_seg()(*args)


def _cnt128(*args):
  return _make_cnt()(*args)


_GRID = 10
_R = N_PAD // _GRID


def _proj_body(x_ref, wl_ref, wr_ref, b_ref, p_ref, q_ref):
  xb = x_ref[...]
  p_ref[...] = jnp.dot(xb, wl_ref[...], preferred_element_type=jnp.float32)
  q_ref[...] = (jnp.dot(xb, wr_ref[...], preferred_element_type=jnp.float32)
                + b_ref[...])


def _combine_body(part_ref, cnt_ref, q_ref, wl_ref, wr_ref, b_ref,
                  p_ref, qn_ref):
  agg = part_ref[0] + part_ref[1]
  cnt = cnt_ref[0, :, 0:1] + cnt_ref[1, :, 0:1]
  inv = 1.0 / jnp.maximum(cnt, 1.0)
  h = jnp.maximum(agg * inv + q_ref[...], 0.0)
  p_ref[...] = jnp.dot(h, wl_ref[...], preferred_element_type=jnp.float32)
  qn_ref[...] = (jnp.dot(h, wr_ref[...], preferred_element_type=jnp.float32)
                 + b_ref[...])


def _last_body(part_ref, cnt_ref, q_ref, wr_ref, b_ref, h_ref, qn_ref):
  agg = part_ref[0] + part_ref[1]
  cnt = cnt_ref[0, :, 0:1] + cnt_ref[1, :, 0:1]
  inv = 1.0 / jnp.maximum(cnt, 1.0)
  h = jnp.maximum(agg * inv + q_ref[...], 0.0)
  h_ref[...] = h
  qn_ref[...] = (jnp.dot(h, wr_ref[...], preferred_element_type=jnp.float32)
                 + b_ref[...])


def _final_body(part_ref, cnt_ref, q_ref, wl_ref, o_ref):
  agg = part_ref[0] + part_ref[1]
  cnt = cnt_ref[0, :, 0:1] + cnt_ref[1, :, 0:1]
  inv = 1.0 / jnp.maximum(cnt, 1.0)
  h = (jnp.dot(agg * inv, wl_ref[...], preferred_element_type=jnp.float32)
       + q_ref[...])
  n = jnp.sqrt(jnp.sum(h * h, axis=1, keepdims=True))
  o_ref[...] = h / jnp.maximum(n, 1e-12)


def _proj_call(xp, wlT, wrT, b, dout):
  return pl.pallas_call(
      _proj_body,
      grid=(_GRID,),
      in_specs=[
          pl.BlockSpec((_R, 128), lambda i: (i, 0)),
          pl.BlockSpec((128, dout), lambda i: (0, 0)),
          pl.BlockSpec((128, dout), lambda i: (0, 0)),
          pl.BlockSpec((1, dout), lambda i: (0, 0)),
      ],
      out_specs=[
          pl.BlockSpec((_R, dout), lambda i: (i, 0)),
          pl.BlockSpec((_R, dout), lambda i: (i, 0)),
      ],
      out_shape=[
          jax.ShapeDtypeStruct((N_PAD, dout), jnp.float32),
          jax.ShapeDtypeStruct((N_PAD, dout), jnp.float32),
      ],
  )(xp, wlT, wrT, b)


def _combine_call(part, cnt, q, wlT, wrT, b, din, dout):
  return pl.pallas_call(
      _combine_body,
      grid=(_GRID,),
      in_specs=[
          pl.BlockSpec((NC, _R, din), lambda i: (0, i, 0)),
          pl.BlockSpec((NC, _R, 128), lambda i: (0, i, 0)),
          pl.BlockSpec((_R, din), lambda i: (i, 0)),
          pl.BlockSpec((din, dout), lambda i: (0, 0)),
          pl.BlockSpec((din, dout), lambda i: (0, 0)),
          pl.BlockSpec((1, dout), lambda i: (0, 0)),
      ],
      out_specs=[
          pl.BlockSpec((_R, dout), lambda i: (i, 0)),
          pl.BlockSpec((_R, dout), lambda i: (i, 0)),
      ],
      out_shape=[
          jax.ShapeDtypeStruct((N_PAD, dout), jnp.float32),
          jax.ShapeDtypeStruct((N_PAD, dout), jnp.float32),
      ],
  )(part, cnt, q, wlT, wrT, b)


def _last_call(part, cnt, q, wrT, b):
  return pl.pallas_call(
      _last_body,
      grid=(_GRID,),
      in_specs=[
          pl.BlockSpec((NC, _R, 128), lambda i: (0, i, 0)),
          pl.BlockSpec((NC, _R, 128), lambda i: (0, i, 0)),
          pl.BlockSpec((_R, 128), lambda i: (i, 0)),
          pl.BlockSpec((128, 64), lambda i: (0, 0)),
          pl.BlockSpec((1, 64), lambda i: (0, 0)),
      ],
      out_specs=[
          pl.BlockSpec((_R, 128), lambda i: (i, 0)),
          pl.BlockSpec((_R, 64), lambda i: (i, 0)),
      ],
      out_shape=[
          jax.ShapeDtypeStruct((N_PAD, 128), jnp.float32),
          jax.ShapeDtypeStruct((N_PAD, 64), jnp.float32),
      ],
  )(part, cnt, q, wrT, b)


def _final_call(part, cnt, q, wlT):
  return pl.pallas_call(
      _final_body,
      grid=(_GRID,),
      in_specs=[
          pl.BlockSpec((NC, _R, 128), lambda i: (0, i, 0)),
          pl.BlockSpec((NC, _R, 128), lambda i: (0, i, 0)),
          pl.BlockSpec((_R, 64), lambda i: (i, 0)),
          pl.BlockSpec((128, 64), lambda i: (0, 0)),
      ],
      out_specs=pl.BlockSpec((_R, 64), lambda i: (i, 0)),
      out_shape=jax.ShapeDtypeStruct((N_PAD, 64), jnp.float32),
  )(part, cnt, q, wlT)


@jax.jit
def kernel(x, edge_index, W1l, b1l, W1r, W2l, b2l, W2r, W3l, b3l, W3r):
  xp = jnp.pad(x, ((0, N_PAD - N_NODES), (0, 0)))
  pad_w = IDXC * CHUNK - EPT
  ei = jnp.reshape(edge_index, (2, NW, EPT))
  ei = jnp.pad(ei, ((0, 0), (0, 0), (0, pad_w)),
               constant_values=N_PAD - 1)
  packed = jnp.reshape(
      jnp.transpose(jnp.reshape(ei, (2, NW, IDXC, CHUNK)), (1, 2, 0, 3)),
      (NW * IDXC, 2, CHUNK))
  z128 = jnp.zeros((N_PAD, 128), jnp.float32)
  ones = jnp.ones((CHUNK, 128), jnp.float32)

  cnt = _cnt128(packed, z128, ones)
  p1, q1 = _proj_call(xp, W1l.T, W1r.T, b1l[None], 128)
  part1 = _seg128(p1, packed, z128)
  p2, q2 = _combine_call(part1, cnt, q1, W2l.T, W2r.T, b2l[None], 128, 128)
  part2 = _seg128(p2, packed, z128)
  h2, q3 = _last_call(part2, cnt, q2, W3r.T, b3l[None])
  part3 = _seg128(h2, packed, z128)
  out = _final_call(part3, cnt, q3, W3l.T)
  return out[:N_NODES]
